# Initial kernel scaffold; baseline (speedup 1.0000x reference)
import jax
import jax.numpy as jnp
from jax.experimental import pallas as pl


def kernel(x, emb0, emb1, W1, b1, W2, b2):
    raise NotImplementedError("write your pallas kernel here")


# speedup vs baseline: 3.1462x; 3.1462x over previous
import functools

import jax
import jax.numpy as jnp
from jax import lax
from jax.experimental import pallas as pl
from jax.experimental.pallas import tpu as pltpu
from jax.experimental.pallas import tpu_sc as plsc

P = 128
NPAD = 16
B = 16384


def _table_kernel(e0_ref, e1_ref, w1_ref, b1_ref, w2_ref, b2_ref, t_ref):
    a0 = jnp.dot(e0_ref[...], w1_ref[0:P, :], preferred_element_type=jnp.float32)
    a1 = jnp.dot(e1_ref[...], w1_ref[P:2 * P, :], preferred_element_type=jnp.float32)
    h = a0[:, None, :] + a1[None, :, :] + b1_ref[...][None, :, :]
    h = jnp.maximum(h, 0.0)
    t = jnp.dot(h.reshape(NPAD * NPAD, P), w2_ref[...],
                preferred_element_type=jnp.float32) + b2_ref[0, 0]
    t_ref[...] = t


def _build_table(emb0, emb1, W1, b1, W2, b2):
    e0 = jnp.zeros((NPAD, P), jnp.float32).at[: emb0.shape[0]].set(emb0)
    e1 = jnp.zeros((NPAD, P), jnp.float32).at[: emb1.shape[0]].set(emb1)
    t = pl.pallas_call(
        _table_kernel,
        out_shape=jax.ShapeDtypeStruct((NPAD * NPAD, 1), jnp.float32),
    )(e0, e1, W1, b1.reshape(1, P), W2, b2.reshape(1, 1))
    return t.reshape(NPAD * NPAD)


def _make_sc_gather():
    info = plsc.get_sparse_core_info()
    nw = info.num_cores * info.num_subcores
    b_per_w = B // nw
    n_grp = b_per_w // 16

    mesh = plsc.VectorSubcoreMesh(core_axis_name="c", subcore_axis_name="s")

    @functools.partial(
        pl.kernel,
        mesh=mesh,
        out_type=jax.ShapeDtypeStruct((B,), jnp.float32),
        compiler_params=pltpu.CompilerParams(needs_layout_passes=False),
        scratch_types=[
            pltpu.VMEM((2 * b_per_w,), jnp.int32),
            pltpu.VMEM((NPAD * NPAD,), jnp.float32),
            pltpu.VMEM((b_per_w,), jnp.float32),
        ],
    )
    def gather_k(x_hbm, t_hbm, out_hbm, x_v, t_v, o_v):
        wid = lax.axis_index("s") * info.num_cores + lax.axis_index("c")
        base = wid * b_per_w
        pltpu.sync_copy(x_hbm.at[pl.ds(2 * base, 2 * b_per_w)], x_v)
        pltpu.sync_copy(t_hbm, t_v)
        lanes2 = lax.iota(jnp.int32, 16) * 2
        for g in range(n_grp):
            ev = lanes2 + (g * 32)
            x0 = plsc.load_gather(x_v, [ev])
            x1 = plsc.load_gather(x_v, [ev + 1])
            o_v[pl.ds(g * 16, 16)] = plsc.load_gather(t_v, [x0 * NPAD + x1])
        pltpu.sync_copy(o_v, out_hbm.at[pl.ds(base, b_per_w)])

    return gather_k


def kernel(x, emb0, emb1, W1, b1, W2, b2):
    t = _build_table(emb0, emb1, W1, b1, W2, b2)
    out = _make_sc_gather()(x.reshape(2 * B), t)
    return out.reshape(B, 1)


# speedup vs baseline: 3.2869x; 1.0447x over previous
import functools

import jax
import jax.numpy as jnp
from jax import lax
from jax.experimental import pallas as pl
from jax.experimental.pallas import tpu as pltpu
from jax.experimental.pallas import tpu_sc as plsc

P = 128
NROW = 10
NPAD = 16
B = 16384


def _table_kernel(e0_ref, e1_ref, w1_ref, b1_ref, w2_ref, b2_ref, t_ref):
    a0 = jnp.dot(e0_ref[...], w1_ref[0:P, :], preferred_element_type=jnp.float32)
    a1 = jnp.dot(e1_ref[...], w1_ref[P:2 * P, :], preferred_element_type=jnp.float32)
    a1p = jnp.concatenate([a1, jnp.zeros((NPAD - NROW, P), jnp.float32)], axis=0)
    h = a0[:, None, :] + a1p[None, :, :] + b1_ref[...][None, :, :]
    h = jnp.maximum(h, 0.0)
    t = jnp.dot(h.reshape(NROW * NPAD, P), w2_ref[...],
                preferred_element_type=jnp.float32) + b2_ref[0, 0]
    t_ref[...] = t


def _build_table(emb0, emb1, W1, b1, W2, b2):
    t = pl.pallas_call(
        _table_kernel,
        out_shape=jax.ShapeDtypeStruct((NROW * NPAD, 1), jnp.float32),
    )(emb0, emb1, W1, b1.reshape(1, P), W2, b2.reshape(1, 1))
    return t.reshape(NROW * NPAD)


def _make_sc_gather():
    info = plsc.get_sparse_core_info()
    nw = info.num_cores * info.num_subcores
    b_per_w = B // nw
    n_grp = b_per_w // 16

    mesh = plsc.VectorSubcoreMesh(core_axis_name="c", subcore_axis_name="s")

    @functools.partial(
        pl.kernel,
        mesh=mesh,
        out_type=jax.ShapeDtypeStruct((B,), jnp.float32),
        compiler_params=pltpu.CompilerParams(needs_layout_passes=False),
        scratch_types=[
            pltpu.VMEM((2 * b_per_w,), jnp.int32),
            pltpu.VMEM((NROW * NPAD,), jnp.float32),
            pltpu.VMEM((b_per_w,), jnp.float32),
            pltpu.SemaphoreType.DMA,
            pltpu.SemaphoreType.DMA,
        ],
    )
    def gather_k(x_hbm, t_hbm, out_hbm, x_v, t_v, o_v, sem_x, sem_t):
        wid = lax.axis_index("s") * info.num_cores + lax.axis_index("c")
        base = wid * b_per_w
        cp_x = pltpu.async_copy(x_hbm.at[pl.ds(2 * base, 2 * b_per_w)], x_v, sem_x)
        cp_t = pltpu.async_copy(t_hbm, t_v, sem_t)
        cp_x.wait()
        cp_t.wait()
        lanes2 = lax.iota(jnp.int32, 16) * 2
        for g in range(n_grp):
            ev = lanes2 + (g * 32)
            x0 = plsc.load_gather(x_v, [ev])
            x1 = plsc.load_gather(x_v, [ev + 1])
            o_v[pl.ds(g * 16, 16)] = plsc.load_gather(t_v, [x0 * NPAD + x1])
        pltpu.sync_copy(o_v, out_hbm.at[pl.ds(base, b_per_w)])

    return gather_k


def kernel(x, emb0, emb1, W1, b1, W2, b2):
    t = _build_table(emb0, emb1, W1, b1, W2, b2)
    out = _make_sc_gather()(x.reshape(2 * B), t)
    return out.reshape(B, 1)


# speedup vs baseline: 3.6437x; 1.1085x over previous
import functools

import jax
import jax.numpy as jnp
from jax import lax
from jax.experimental import pallas as pl
from jax.experimental.pallas import tpu as pltpu
from jax.experimental.pallas import tpu_sc as plsc

P = 128
NROW = 10
NPAD = 16
B = 16384


def _table_kernel(e0_ref, e1_ref, w1_ref, b1_ref, w2_ref, b2_ref, t_ref):
    a0 = jnp.dot(e0_ref[...], w1_ref[0:P, :], preferred_element_type=jnp.float32)
    a1 = jnp.dot(e1_ref[...], w1_ref[P:2 * P, :], preferred_element_type=jnp.float32)
    a1p = jnp.concatenate([a1, jnp.zeros((NPAD - NROW, P), jnp.float32)], axis=0)
    h = a0[:, None, :] + a1p[None, :, :] + b1_ref[...][None, :, :]
    h = jnp.maximum(h, 0.0)
    t = jnp.dot(h.reshape(NROW * NPAD, P), w2_ref[...],
                preferred_element_type=jnp.float32) + b2_ref[0, 0]
    t_ref[...] = t


def _build_table(emb0, emb1, W1, b1, W2, b2):
    t = pl.pallas_call(
        _table_kernel,
        out_shape=jax.ShapeDtypeStruct((NROW * NPAD, 1), jnp.float32),
    )(emb0, emb1, W1, b1.reshape(1, P), W2, b2.reshape(1, 1))
    return t.reshape(NROW * NPAD)


def _make_sc_gather():
    info = plsc.get_sparse_core_info()
    nw = info.num_cores * info.num_subcores
    b_per_w = B // nw
    n_grp = b_per_w // 16

    mesh = plsc.VectorSubcoreMesh(core_axis_name="c", subcore_axis_name="s")

    @functools.partial(
        pl.kernel,
        mesh=mesh,
        out_type=jax.ShapeDtypeStruct((B,), jnp.float32),
        compiler_params=pltpu.CompilerParams(needs_layout_passes=False),
        scratch_types=[
            pltpu.VMEM((b_per_w, 2), jnp.int32),
            pltpu.VMEM((NROW * NPAD,), jnp.float32),
            pltpu.VMEM((b_per_w,), jnp.float32),
            pltpu.SemaphoreType.DMA,
            pltpu.SemaphoreType.DMA,
        ],
    )
    def gather_k(x_hbm, t_hbm, out_hbm, x_v, t_v, o_v, sem_x, sem_t):
        wid = lax.axis_index("s") * info.num_cores + lax.axis_index("c")
        base = wid * b_per_w
        cp_x = pltpu.async_copy(x_hbm.at[pl.ds(base, b_per_w)], x_v, sem_x)
        cp_t = pltpu.async_copy(t_hbm, t_v, sem_t)
        cp_x.wait()
        cp_t.wait()
        lanes = lax.iota(jnp.int32, 16)
        zeros = jnp.zeros((16,), jnp.int32)
        ones = zeros + 1
        for g in range(n_grp):
            rows = lanes + (g * 16)
            x0 = plsc.load_gather(x_v, [rows, zeros])
            x1 = plsc.load_gather(x_v, [rows, ones])
            o_v[pl.ds(g * 16, 16)] = plsc.load_gather(t_v, [x0 * NPAD + x1])
        pltpu.sync_copy(o_v, out_hbm.at[pl.ds(base, b_per_w)])

    return gather_k


def kernel(x, emb0, emb1, W1, b1, W2, b2):
    t = _build_table(emb0, emb1, W1, b1, W2, b2)
    out = _make_sc_gather()(x, t)
    return out.reshape(B, 1)


# speedup vs baseline: 5.6689x; 1.5558x over previous
import functools

import jax
import jax.numpy as jnp
from jax import lax
from jax.experimental import pallas as pl
from jax.experimental.pallas import tpu as pltpu
from jax.experimental.pallas import tpu_sc as plsc

P = 128
NROW = 10
NPAD = 16
B = 16384
BLK = 128


def _table_kernel(e0_ref, e1_ref, w1_ref, b1_ref, w2_ref, b2_ref, t_ref):
    a0 = jnp.dot(e0_ref[...], w1_ref[0:P, :], preferred_element_type=jnp.float32)
    a1 = jnp.dot(e1_ref[...], w1_ref[P:2 * P, :], preferred_element_type=jnp.float32)
    a1p = jnp.concatenate([a1, jnp.zeros((NPAD - NROW, P), jnp.float32)], axis=0)
    h = a0[:, None, :] + a1p[None, :, :] + b1_ref[...][None, :, :]
    h = jnp.maximum(h, 0.0)
    t = jnp.sum(h.reshape(NROW * NPAD, P) * w2_ref[...], axis=1) + b2_ref[0, 0]
    t_ref[...] = t


def _build_table(emb0, emb1, W1, b1, W2, b2):
    return pl.pallas_call(
        _table_kernel,
        out_shape=jax.ShapeDtypeStruct((NROW * NPAD,), jnp.float32),
    )(emb0, emb1, W1, b1.reshape(1, P), W2.T, b2.reshape(1, 1))


def _make_sc_gather():
    info = plsc.get_sparse_core_info()
    nw = info.num_cores * info.num_subcores
    b_per_w = B // nw
    k_per_w = b_per_w // BLK

    mesh = plsc.VectorSubcoreMesh(core_axis_name="c", subcore_axis_name="s")

    @functools.partial(
        pl.kernel,
        mesh=mesh,
        out_type=jax.ShapeDtypeStruct((B,), jnp.float32),
        compiler_params=pltpu.CompilerParams(
            needs_layout_passes=False, use_tc_tiling_on_sc=False
        ),
        scratch_types=[
            pltpu.VMEM((k_per_w, 2, BLK), jnp.int32),
            pltpu.VMEM((NROW * NPAD,), jnp.float32),
            pltpu.VMEM((b_per_w,), jnp.float32),
            pltpu.SemaphoreType.DMA,
            pltpu.SemaphoreType.DMA,
        ],
    )
    def gather_k(xb_hbm, t_hbm, out_hbm, x_v, t_v, o_v, sem_x, sem_t):
        wid = lax.axis_index("s") * info.num_cores + lax.axis_index("c")
        base = wid * k_per_w
        cp_x = pltpu.async_copy(xb_hbm.at[pl.ds(base, k_per_w)], x_v, sem_x)
        cp_t = pltpu.async_copy(t_hbm, t_v, sem_t)
        cp_x.wait()
        cp_t.wait()
        for blk in range(k_per_w):
            for s in range(BLK // 16):
                x0 = x_v[blk, 0, pl.ds(s * 16, 16)]
                x1 = x_v[blk, 1, pl.ds(s * 16, 16)]
                vals = plsc.load_gather(t_v, [x0 * NPAD + x1])
                o_v[pl.ds(blk * BLK + s * 16, 16)] = vals
        pltpu.sync_copy(o_v, out_hbm.at[pl.ds(wid * b_per_w, b_per_w)])

    return gather_k


def kernel(x, emb0, emb1, W1, b1, W2, b2):
    t = _build_table(emb0, emb1, W1, b1, W2, b2)
    xb = x.reshape(B // BLK, BLK, 2).transpose(0, 2, 1)
    out = _make_sc_gather()(xb, t)
    return out.reshape(B, 1)
